# Initial kernel scaffold; baseline (speedup 1.0000x reference)
import jax
import jax.numpy as jnp
from jax.experimental import pallas as pl


def kernel(x, e_idx, b, W0, c0, W1, c1, W2, c2, S0w, S0c, S1w, S1c, L1w, L1c, L2w, L2c):
    raise NotImplementedError("write your pallas kernel here")


# speedup vs baseline: 5.3776x; 5.3776x over previous
import functools

import jax
import jax.numpy as jnp
from jax import lax
from jax.experimental import pallas as pl
from jax.experimental.pallas import tpu as pltpu
from jax.experimental.pallas import tpu_sc as plsc

_N = 10000
_E = 320000
_NG = 256
_DF = 128
_H = 64
_NC = 2
_NS = 16
_NW = _NC * _NS
_NPAD = 10240
_RPW = _NPAD // _NW
_ETOT = _E + _N
_GSZ = 256
_NGRP = 44
_EPW = _NGRP * _GSZ
_EPAD = _NW * _EPW
_DUMMY = _N
_DG = _NG
_PG = 320

_mesh = plsc.VectorSubcoreMesh(core_axis_name="c", subcore_axis_name="s")
_sc_params = pltpu.CompilerParams(use_tc_tiling_on_sc=False)


def _zero_2d(ref):
    def body(i, _):
        ref[i >> 2, pl.ds((i & 3) * 16, 16)] = jnp.zeros((16,), jnp.float32)
        return 0
    lax.fori_loop(0, 512, body, 0)


def _fill_1d(ref, n16, val):
    def body(i, _):
        ref[pl.ds(i * 16, 16)] = jnp.full((16,), val, jnp.float32)
        return 0
    lax.fori_loop(0, n16, body, 0)


@functools.partial(
    pl.kernel,
    out_type=[
        jax.ShapeDtypeStruct((_NC, _NPAD), jnp.float32),
        jax.ShapeDtypeStruct((_NC, _PG), jnp.float32),
    ],
    mesh=_mesh,
    compiler_params=_sc_params,
    scratch_types=[
        pltpu.VMEM((_EPW // 128, 128), jnp.int32),
        pltpu.VMEM((5, 64), jnp.int32),
        pltpu.VMEM((128,), jnp.float32),
        pltpu.VMEM((640,), jnp.float32),
        pltpu.VMEM_SHARED((_NPAD,), jnp.float32),
        pltpu.VMEM_SHARED((_PG,), jnp.float32),
    ],
)
def _sc_deg_cnt(dstc, b3, deg_out, cnt_out, dsti, bi, ones_v, zv, deg_sh, cnt_sh):
    cid = lax.axis_index("c")
    sid = lax.axis_index("s")
    wid = cid * _NS + sid
    _fill_1d(zv, 40, 0.0)
    _fill_1d(ones_v, 8, 1.0)
    pltpu.sync_copy(zv, deg_sh.at[pl.ds(sid * 640, 640)])

    @pl.when(sid == 0)
    def _():
        pltpu.sync_copy(zv.at[pl.ds(0, _PG)], cnt_sh)

    plsc.subcore_barrier()
    pltpu.sync_copy(dstc.at[wid], dsti)
    pltpu.sync_copy(b3.at[wid], bi)

    def body(j, _):
        pltpu.sync_copy(ones_v, deg_sh.at[dsti.at[j]], add=True)
        return 0
    lax.fori_loop(0, _EPW // 128, body, 0)
    for k in range(5):
        pltpu.sync_copy(ones_v.at[pl.ds(0, 64)], cnt_sh.at[bi.at[k]], add=True)
    plsc.subcore_barrier()
    pltpu.sync_copy(deg_sh.at[pl.ds(sid * 640, 640)],
                    deg_out.at[cid, pl.ds(sid * 640, 640)])

    @pl.when(sid == 0)
    def _():
        pltpu.sync_copy(cnt_sh, cnt_out.at[cid])


@functools.partial(
    pl.kernel,
    out_type=jax.ShapeDtypeStruct((_NC, _NPAD, _H), jnp.float32),
    mesh=_mesh,
    compiler_params=_sc_params,
    scratch_types=[
        pltpu.VMEM((_EPW,), jnp.int32),
        pltpu.VMEM((_NGRP, _GSZ), jnp.int32),
        pltpu.VMEM((_GSZ, _H), jnp.float32),
        pltpu.VMEM((_GSZ, _H), jnp.float32),
        pltpu.VMEM_SHARED((_NPAD, _H), jnp.float32),
        pltpu.SemaphoreType.DMA,
        pltpu.SemaphoreType.DMA,
    ],
)
def _sc_agg(hw, src2, dst3, out, srci, dsti, buf_a, buf_b, acc_sh, sem_a, sem_b):
    cid = lax.axis_index("c")
    sid = lax.axis_index("s")
    wid = cid * _NS + sid

    def zbody(i, _):
        buf_a[i >> 2, pl.ds((i & 3) * 16, 16)] = jnp.zeros((16,), jnp.float32)
        return 0
    lax.fori_loop(0, _GSZ * 4, zbody, 0)
    for k in range(2):
        pltpu.sync_copy(buf_a, acc_sh.at[pl.ds(sid * 640 + k * _GSZ, _GSZ)])
    pltpu.sync_copy(buf_a.at[pl.ds(0, 128)],
                    acc_sh.at[pl.ds(sid * 640 + 512, 128)])
    plsc.subcore_barrier()
    pltpu.sync_copy(src2.at[wid], srci)
    pltpu.sync_copy(dst3.at[wid], dsti)

    pltpu.async_copy(hw.at[srci.at[pl.ds(0, _GSZ)]], buf_a, sem_a)

    def body(i, _):
        j0 = 2 * i
        j1 = j0 + 1
        pltpu.async_copy(hw.at[srci.at[pl.ds(j1 * _GSZ, _GSZ)]], buf_b, sem_b)
        pltpu.make_async_copy(hw.at[srci.at[pl.ds(j0 * _GSZ, _GSZ)]], buf_a,
                              sem_a).wait()
        pltpu.sync_copy(buf_a, acc_sh.at[dsti.at[j0]], add=True)

        @pl.when(j0 + 2 < _NGRP)
        def _():
            pltpu.async_copy(hw.at[srci.at[pl.ds((j0 + 2) * _GSZ, _GSZ)]],
                             buf_a, sem_a)

        pltpu.make_async_copy(hw.at[srci.at[pl.ds(j1 * _GSZ, _GSZ)]], buf_b,
                              sem_b).wait()
        pltpu.sync_copy(buf_b, acc_sh.at[dsti.at[j1]], add=True)
        return 0
    lax.fori_loop(0, _NGRP // 2, body, 0)
    plsc.subcore_barrier()
    pltpu.sync_copy(acc_sh.at[pl.ds(sid * 640, 640)],
                    out.at[cid, pl.ds(sid * 640, 640)])


@functools.partial(
    pl.kernel,
    out_type=jax.ShapeDtypeStruct((_NC, _PG, _H), jnp.float32),
    mesh=_mesh,
    compiler_params=_sc_params,
    scratch_types=[
        pltpu.VMEM((5, 64), jnp.int32),
        pltpu.VMEM((_RPW, _H), jnp.float32),
        pltpu.VMEM((128, _H), jnp.float32),
        pltpu.VMEM_SHARED((_PG, _H), jnp.float32),
    ],
)
def _sc_pool(h2, b3, out, bi, hrows, zr, pool_sh):
    cid = lax.axis_index("c")
    sid = lax.axis_index("s")
    wid = cid * _NS + sid
    _zero_2d(zr)

    @pl.when(sid == 0)
    def _():
        pltpu.sync_copy(zr, pool_sh.at[pl.ds(0, 128)])
        pltpu.sync_copy(zr, pool_sh.at[pl.ds(128, 128)])
        pltpu.sync_copy(zr.at[pl.ds(0, 64)], pool_sh.at[pl.ds(256, 64)])

    plsc.subcore_barrier()
    pltpu.sync_copy(b3.at[wid], bi)
    pltpu.sync_copy(h2.at[pl.ds(wid * _RPW, _RPW)], hrows)
    for k in range(5):
        pltpu.sync_copy(hrows.at[pl.ds(k * 64, 64)], pool_sh.at[bi.at[k]], add=True)
    plsc.subcore_barrier()

    @pl.when(sid == 0)
    def _():
        pltpu.sync_copy(pool_sh, out.at[cid])


def _tc1_body(x_ref, w_ref, degp_ref, hw_ref, dinv_ref):
    deg = degp_ref[0, :] + degp_ref[1, :]
    dinv = lax.rsqrt(jnp.maximum(deg, 1.0))
    dinv_ref[0, :] = dinv
    hw = jnp.dot(x_ref[...], w_ref[...], preferred_element_type=jnp.float32)
    hw_ref[...] = hw * dinv[:, None]


_tc1 = pl.pallas_call(
    _tc1_body,
    out_shape=[
        jax.ShapeDtypeStruct((_NPAD, _H), jnp.float32),
        jax.ShapeDtypeStruct((1, _NPAD), jnp.float32),
    ],
)


def _tc_mid_body(with_hs, *refs):
    if with_hs:
        pp_ref, dinv_ref, hs_ref, w_ref, sw_ref, c_ref, sc_ref, hw_ref, hsout_ref = refs
    else:
        pp_ref, dinv_ref, w_ref, sw_ref, c_ref, sc_ref, hw_ref, hsout_ref = refs
        hs_ref = None
    dinv = dinv_ref[0, :]
    h = jnp.maximum(dinv[:, None] * (pp_ref[0] + pp_ref[1]) + c_ref[0, :], 0.0)
    if hs_ref is not None:
        h = h + hs_ref[...]
    hw = jnp.dot(h, w_ref[...], preferred_element_type=jnp.float32)
    hw_ref[...] = hw * dinv[:, None]
    hsout_ref[...] = jnp.dot(h, sw_ref[...], preferred_element_type=jnp.float32) + sc_ref[0, :]


_tc_mid_shapes = [
    jax.ShapeDtypeStruct((_NPAD, _H), jnp.float32),
    jax.ShapeDtypeStruct((_NPAD, _H), jnp.float32),
]
_tc2 = pl.pallas_call(functools.partial(_tc_mid_body, False), out_shape=_tc_mid_shapes)
_tc3 = pl.pallas_call(functools.partial(_tc_mid_body, True), out_shape=_tc_mid_shapes)


def _tc4_body(pp_ref, dinv_ref, hs_ref, c_ref, h2_ref):
    dinv = dinv_ref[0, :]
    h2_ref[...] = (
        jnp.maximum(dinv[:, None] * (pp_ref[0] + pp_ref[1]) + c_ref[0, :], 0.0)
        + hs_ref[...]
    )


_tc4 = pl.pallas_call(
    _tc4_body, out_shape=jax.ShapeDtypeStruct((_NPAD, _H), jnp.float32))


def _tc5_body(poolp_ref, cntp_ref, l1w_ref, l1c_ref, l2w_ref, l2c_ref, out_ref):
    sums = poolp_ref[0] + poolp_ref[1]
    cnt = cntp_ref[0, :] + cntp_ref[1, :]
    pooled = sums[: _NG] / jnp.maximum(cnt[: _NG], 1.0)[:, None]
    z = jnp.maximum(
        jnp.dot(pooled, l1w_ref[...], preferred_element_type=jnp.float32)
        + l1c_ref[0, :], 0.0)
    out_ref[...] = (
        jnp.dot(z, l2w_ref[...], preferred_element_type=jnp.float32) + l2c_ref[0, :])


_tc5 = pl.pallas_call(
    _tc5_body, out_shape=jax.ShapeDtypeStruct((_NG, 1), jnp.float32))


def kernel(x, e_idx, b, W0, c0, W1, c1, W2, c2, S0w, S0c, S1w, S1c, L1w, L1c, L2w, L2c):
    loop = jnp.arange(_N, dtype=jnp.int32)
    pad = jnp.full((_EPAD - _ETOT,), _DUMMY, jnp.int32)
    src2 = jnp.concatenate([e_idx[0], loop, pad]).reshape(_NW, _EPW)
    dst_flat = jnp.concatenate([e_idx[1], loop, pad])
    dst3 = dst_flat.reshape(_NW, _NGRP, _GSZ)
    dstc = dst_flat.reshape(_NW, _EPW // 128, 128)
    xp = jnp.concatenate([x, jnp.zeros((_NPAD - _N, _DF), jnp.float32)])
    b3 = jnp.concatenate([b, jnp.full((_NPAD - _N,), _DG, jnp.int32)]).reshape(
        _NW, 5, 64)

    degp, cntp = _sc_deg_cnt(dstc, b3)
    hw0, dinv = _tc1(xp, W0, degp)
    p0 = _sc_agg(hw0, src2, dst3)
    hw1, hs0 = _tc2(p0, dinv, W1, S0w, c0.reshape(1, _H), S0c.reshape(1, _H))
    p1 = _sc_agg(hw1, src2, dst3)
    hw2, hs1 = _tc3(p1, dinv, hs0, W2, S1w, c1.reshape(1, _H), S1c.reshape(1, _H))
    p2 = _sc_agg(hw2, src2, dst3)
    h2 = _tc4(p2, dinv, hs1, c2.reshape(1, _H))
    poolp = _sc_pool(h2, b3)
    return _tc5(poolp, cntp, L1w, L1c.reshape(1, _H // 2), L2w, L2c.reshape(1, 1))


# speedup vs baseline: 5.3892x; 1.0022x over previous
import functools

import jax
import jax.numpy as jnp
from jax import lax
from jax.experimental import pallas as pl
from jax.experimental.pallas import tpu as pltpu
from jax.experimental.pallas import tpu_sc as plsc

_N = 10000
_E = 320000
_NG = 256
_DF = 128
_H = 64
_NC = 2
_NS = 16
_NW = _NC * _NS
_NPAD = 10240
_RPW = _NPAD // _NW
_ETOT = _E + _N
_GSZ = 128
_NGRP = 88
_EPW = _NGRP * _GSZ
_EPAD = _NW * _EPW
_DUMMY = _N
_DG = _NG
_PG = 320

_mesh = plsc.VectorSubcoreMesh(core_axis_name="c", subcore_axis_name="s")
_sc_params = pltpu.CompilerParams(use_tc_tiling_on_sc=False)


def _zero_2d(ref):
    def body(i, _):
        ref[i >> 2, pl.ds((i & 3) * 16, 16)] = jnp.zeros((16,), jnp.float32)
        return 0
    lax.fori_loop(0, 512, body, 0)


def _fill_1d(ref, n16, val):
    def body(i, _):
        ref[pl.ds(i * 16, 16)] = jnp.full((16,), val, jnp.float32)
        return 0
    lax.fori_loop(0, n16, body, 0)


@functools.partial(
    pl.kernel,
    out_type=[
        jax.ShapeDtypeStruct((_NC, _NPAD), jnp.float32),
        jax.ShapeDtypeStruct((_NC, _PG), jnp.float32),
    ],
    mesh=_mesh,
    compiler_params=_sc_params,
    scratch_types=[
        pltpu.VMEM((_EPW // 128, 128), jnp.int32),
        pltpu.VMEM((5, 64), jnp.int32),
        pltpu.VMEM((128,), jnp.float32),
        pltpu.VMEM((640,), jnp.float32),
        pltpu.VMEM_SHARED((_NPAD,), jnp.float32),
        pltpu.VMEM_SHARED((_PG,), jnp.float32),
    ],
)
def _sc_deg_cnt(dstc, b3, deg_out, cnt_out, dsti, bi, ones_v, zv, deg_sh, cnt_sh):
    cid = lax.axis_index("c")
    sid = lax.axis_index("s")
    wid = cid * _NS + sid
    _fill_1d(zv, 40, 0.0)
    _fill_1d(ones_v, 8, 1.0)
    pltpu.sync_copy(zv, deg_sh.at[pl.ds(sid * 640, 640)])

    @pl.when(sid == 0)
    def _():
        pltpu.sync_copy(zv.at[pl.ds(0, _PG)], cnt_sh)

    plsc.subcore_barrier()
    pltpu.sync_copy(dstc.at[wid], dsti)
    pltpu.sync_copy(b3.at[wid], bi)

    def body(j, _):
        pltpu.sync_copy(ones_v, deg_sh.at[dsti.at[j]], add=True)
        return 0
    lax.fori_loop(0, _EPW // 128, body, 0)
    for k in range(5):
        pltpu.sync_copy(ones_v.at[pl.ds(0, 64)], cnt_sh.at[bi.at[k]], add=True)
    plsc.subcore_barrier()
    pltpu.sync_copy(deg_sh.at[pl.ds(sid * 640, 640)],
                    deg_out.at[cid, pl.ds(sid * 640, 640)])

    @pl.when(sid == 0)
    def _():
        pltpu.sync_copy(cnt_sh, cnt_out.at[cid])


@functools.partial(
    pl.kernel,
    out_type=jax.ShapeDtypeStruct((_NC, _NPAD, _H), jnp.float32),
    mesh=_mesh,
    compiler_params=_sc_params,
    scratch_types=[
        pltpu.VMEM((_EPW,), jnp.int32),
        pltpu.VMEM((_NGRP, _GSZ), jnp.int32),
        pltpu.VMEM((_GSZ, _H), jnp.float32),
        pltpu.VMEM((_GSZ, _H), jnp.float32),
        pltpu.VMEM_SHARED((_NPAD, _H), jnp.float32),
        pltpu.SemaphoreType.DMA,
        pltpu.SemaphoreType.DMA,
    ],
)
def _sc_agg(hw, src2, dst3, out, srci, dsti, buf_a, buf_b, acc_sh, sem_a, sem_b):
    cid = lax.axis_index("c")
    sid = lax.axis_index("s")
    wid = cid * _NS + sid

    def zbody(i, _):
        buf_a[i >> 2, pl.ds((i & 3) * 16, 16)] = jnp.zeros((16,), jnp.float32)
        return 0
    lax.fori_loop(0, _GSZ * 4, zbody, 0)
    for k in range(5):
        pltpu.sync_copy(buf_a, acc_sh.at[pl.ds(sid * 640 + k * _GSZ, _GSZ)])
    plsc.subcore_barrier()
    pltpu.sync_copy(src2.at[wid], srci)
    pltpu.sync_copy(dst3.at[wid], dsti)

    pltpu.async_copy(hw.at[srci.at[pl.ds(0, _GSZ)]], buf_a, sem_a)

    def body(i, _):
        j0 = 2 * i
        j1 = j0 + 1
        pltpu.async_copy(hw.at[srci.at[pl.ds(j1 * _GSZ, _GSZ)]], buf_b, sem_b)
        pltpu.make_async_copy(hw.at[srci.at[pl.ds(j0 * _GSZ, _GSZ)]], buf_a,
                              sem_a).wait()
        pltpu.sync_copy(buf_a, acc_sh.at[dsti.at[j0]], add=True)

        @pl.when(j0 + 2 < _NGRP)
        def _():
            pltpu.async_copy(hw.at[srci.at[pl.ds((j0 + 2) * _GSZ, _GSZ)]],
                             buf_a, sem_a)

        pltpu.make_async_copy(hw.at[srci.at[pl.ds(j1 * _GSZ, _GSZ)]], buf_b,
                              sem_b).wait()
        pltpu.sync_copy(buf_b, acc_sh.at[dsti.at[j1]], add=True)
        return 0
    lax.fori_loop(0, _NGRP // 2, body, 0)
    plsc.subcore_barrier()
    pltpu.sync_copy(acc_sh.at[pl.ds(sid * 640, 640)],
                    out.at[cid, pl.ds(sid * 640, 640)])


@functools.partial(
    pl.kernel,
    out_type=jax.ShapeDtypeStruct((_NC, _PG, _H), jnp.float32),
    mesh=_mesh,
    compiler_params=_sc_params,
    scratch_types=[
        pltpu.VMEM((5, 64), jnp.int32),
        pltpu.VMEM((_RPW, _H), jnp.float32),
        pltpu.VMEM((128, _H), jnp.float32),
        pltpu.VMEM_SHARED((_PG, _H), jnp.float32),
    ],
)
def _sc_pool(h2, b3, out, bi, hrows, zr, pool_sh):
    cid = lax.axis_index("c")
    sid = lax.axis_index("s")
    wid = cid * _NS + sid
    _zero_2d(zr)

    @pl.when(sid == 0)
    def _():
        pltpu.sync_copy(zr, pool_sh.at[pl.ds(0, 128)])
        pltpu.sync_copy(zr, pool_sh.at[pl.ds(128, 128)])
        pltpu.sync_copy(zr.at[pl.ds(0, 64)], pool_sh.at[pl.ds(256, 64)])

    plsc.subcore_barrier()
    pltpu.sync_copy(b3.at[wid], bi)
    pltpu.sync_copy(h2.at[pl.ds(wid * _RPW, _RPW)], hrows)
    for k in range(5):
        pltpu.sync_copy(hrows.at[pl.ds(k * 64, 64)], pool_sh.at[bi.at[k]], add=True)
    plsc.subcore_barrier()

    @pl.when(sid == 0)
    def _():
        pltpu.sync_copy(pool_sh, out.at[cid])


def _tc1_body(x_ref, w_ref, degp_ref, hw_ref, dinv_ref):
    deg = degp_ref[0, :] + degp_ref[1, :]
    dinv = lax.rsqrt(jnp.maximum(deg, 1.0))
    dinv_ref[0, :] = dinv
    hw = jnp.dot(x_ref[...], w_ref[...], preferred_element_type=jnp.float32)
    hw_ref[...] = hw * dinv[:, None]


_tc1 = pl.pallas_call(
    _tc1_body,
    out_shape=[
        jax.ShapeDtypeStruct((_NPAD, _H), jnp.float32),
        jax.ShapeDtypeStruct((1, _NPAD), jnp.float32),
    ],
)


def _tc_mid_body(with_hs, *refs):
    if with_hs:
        pp_ref, dinv_ref, hs_ref, w_ref, sw_ref, c_ref, sc_ref, hw_ref, hsout_ref = refs
    else:
        pp_ref, dinv_ref, w_ref, sw_ref, c_ref, sc_ref, hw_ref, hsout_ref = refs
        hs_ref = None
    dinv = dinv_ref[0, :]
    h = jnp.maximum(dinv[:, None] * (pp_ref[0] + pp_ref[1]) + c_ref[0, :], 0.0)
    if hs_ref is not None:
        h = h + hs_ref[...]
    hw = jnp.dot(h, w_ref[...], preferred_element_type=jnp.float32)
    hw_ref[...] = hw * dinv[:, None]
    hsout_ref[...] = jnp.dot(h, sw_ref[...], preferred_element_type=jnp.float32) + sc_ref[0, :]


_tc_mid_shapes = [
    jax.ShapeDtypeStruct((_NPAD, _H), jnp.float32),
    jax.ShapeDtypeStruct((_NPAD, _H), jnp.float32),
]
_tc2 = pl.pallas_call(functools.partial(_tc_mid_body, False), out_shape=_tc_mid_shapes)
_tc3 = pl.pallas_call(functools.partial(_tc_mid_body, True), out_shape=_tc_mid_shapes)


def _tc4_body(pp_ref, dinv_ref, hs_ref, c_ref, h2_ref):
    dinv = dinv_ref[0, :]
    h2_ref[...] = (
        jnp.maximum(dinv[:, None] * (pp_ref[0] + pp_ref[1]) + c_ref[0, :], 0.0)
        + hs_ref[...]
    )


_tc4 = pl.pallas_call(
    _tc4_body, out_shape=jax.ShapeDtypeStruct((_NPAD, _H), jnp.float32))


def _tc5_body(poolp_ref, cntp_ref, l1w_ref, l1c_ref, l2w_ref, l2c_ref, out_ref):
    sums = poolp_ref[0] + poolp_ref[1]
    cnt = cntp_ref[0, :] + cntp_ref[1, :]
    pooled = sums[: _NG] / jnp.maximum(cnt[: _NG], 1.0)[:, None]
    z = jnp.maximum(
        jnp.dot(pooled, l1w_ref[...], preferred_element_type=jnp.float32)
        + l1c_ref[0, :], 0.0)
    out_ref[...] = (
        jnp.dot(z, l2w_ref[...], preferred_element_type=jnp.float32) + l2c_ref[0, :])


_tc5 = pl.pallas_call(
    _tc5_body, out_shape=jax.ShapeDtypeStruct((_NG, 1), jnp.float32))


def kernel(x, e_idx, b, W0, c0, W1, c1, W2, c2, S0w, S0c, S1w, S1c, L1w, L1c, L2w, L2c):
    loop = jnp.arange(_N, dtype=jnp.int32)
    pad = jnp.full((_EPAD - _ETOT,), _DUMMY, jnp.int32)
    src2 = jnp.concatenate([e_idx[0], loop, pad]).reshape(_NW, _EPW)
    dst_flat = jnp.concatenate([e_idx[1], loop, pad])
    dst3 = dst_flat.reshape(_NW, _NGRP, _GSZ)
    dstc = dst_flat.reshape(_NW, _EPW // 128, 128)
    xp = jnp.concatenate([x, jnp.zeros((_NPAD - _N, _DF), jnp.float32)])
    b3 = jnp.concatenate([b, jnp.full((_NPAD - _N,), _DG, jnp.int32)]).reshape(
        _NW, 5, 64)

    degp, cntp = _sc_deg_cnt(dstc, b3)
    hw0, dinv = _tc1(xp, W0, degp)
    p0 = _sc_agg(hw0, src2, dst3)
    hw1, hs0 = _tc2(p0, dinv, W1, S0w, c0.reshape(1, _H), S0c.reshape(1, _H))
    p1 = _sc_agg(hw1, src2, dst3)
    hw2, hs1 = _tc3(p1, dinv, hs0, W2, S1w, c1.reshape(1, _H), S1c.reshape(1, _H))
    p2 = _sc_agg(hw2, src2, dst3)
    h2 = _tc4(p2, dinv, hs1, c2.reshape(1, _H))
    poolp = _sc_pool(h2, b3)
    return _tc5(poolp, cntp, L1w, L1c.reshape(1, _H // 2), L2w, L2c.reshape(1, 1))


# speedup vs baseline: 5.6128x; 1.0415x over previous
import functools

import jax
import jax.numpy as jnp
from jax import lax
from jax.experimental import pallas as pl
from jax.experimental.pallas import tpu as pltpu
from jax.experimental.pallas import tpu_sc as plsc

_N = 10000
_E = 320000
_NG = 256
_DF = 128
_H = 64
_NC = 2
_NS = 16
_NW = _NC * _NS
_NPAD = 10240
_RPW = _NPAD // _NW
_ETOT = _E + _N
_GSZ = 128
_NGRP = 88
_EPW = _NGRP * _GSZ
_EPAD = _NW * _EPW
_DUMMY = _N
_DG = _NG
_PG = 320

_mesh = plsc.VectorSubcoreMesh(core_axis_name="c", subcore_axis_name="s")
_sc_params = pltpu.CompilerParams(use_tc_tiling_on_sc=False)


def _zero_2d(ref):
    def body(i, _):
        ref[i >> 2, pl.ds((i & 3) * 16, 16)] = jnp.zeros((16,), jnp.float32)
        return 0
    lax.fori_loop(0, 512, body, 0)


def _fill_1d(ref, n16, val):
    def body(i, _):
        ref[pl.ds(i * 16, 16)] = jnp.full((16,), val, jnp.float32)
        return 0
    lax.fori_loop(0, n16, body, 0)


@functools.partial(
    pl.kernel,
    out_type=[
        jax.ShapeDtypeStruct((_NC, _NPAD), jnp.float32),
        jax.ShapeDtypeStruct((_NC, _PG), jnp.float32),
    ],
    mesh=_mesh,
    compiler_params=_sc_params,
    scratch_types=[
        pltpu.VMEM((_EPW // 128, 128), jnp.int32),
        pltpu.VMEM((5, 64), jnp.int32),
        pltpu.VMEM((128,), jnp.float32),
        pltpu.VMEM((640,), jnp.float32),
        pltpu.VMEM_SHARED((_NPAD,), jnp.float32),
        pltpu.VMEM_SHARED((_PG,), jnp.float32),
    ],
)
def _sc_deg_cnt(dstc, b3, deg_out, cnt_out, dsti, bi, ones_v, zv, deg_sh, cnt_sh):
    cid = lax.axis_index("c")
    sid = lax.axis_index("s")
    wid = cid * _NS + sid
    _fill_1d(zv, 40, 0.0)
    _fill_1d(ones_v, 8, 1.0)
    pltpu.sync_copy(zv, deg_sh.at[pl.ds(sid * 640, 640)])

    @pl.when(sid == 0)
    def _():
        pltpu.sync_copy(zv.at[pl.ds(0, _PG)], cnt_sh)

    plsc.subcore_barrier()
    pltpu.sync_copy(dstc.at[wid], dsti)
    pltpu.sync_copy(b3.at[wid], bi)

    def body(j, _):
        pltpu.sync_copy(ones_v, deg_sh.at[dsti.at[j]], add=True)
        return 0
    lax.fori_loop(0, _EPW // 128, body, 0)
    for k in range(5):
        pltpu.sync_copy(ones_v.at[pl.ds(0, 64)], cnt_sh.at[bi.at[k]], add=True)
    plsc.subcore_barrier()
    pltpu.sync_copy(deg_sh.at[pl.ds(sid * 640, 640)],
                    deg_out.at[cid, pl.ds(sid * 640, 640)])

    @pl.when(sid == 0)
    def _():
        pltpu.sync_copy(cnt_sh, cnt_out.at[cid])


@functools.partial(
    pl.kernel,
    out_type=jax.ShapeDtypeStruct((_NC, _NPAD, _H), jnp.float32),
    mesh=_mesh,
    compiler_params=_sc_params,
    scratch_types=[
        pltpu.VMEM((_NGRP, _GSZ), jnp.int32),
        pltpu.VMEM((_NGRP, _GSZ), jnp.int32),
        pltpu.VMEM((_GSZ, _H), jnp.float32),
        pltpu.VMEM_SHARED((_NPAD, _H), jnp.float32),
    ],
)
def _sc_agg(hw, src2, dst3, out, srci, dsti, buf_a, acc_sh):
    cid = lax.axis_index("c")
    sid = lax.axis_index("s")
    wid = cid * _NS + sid

    def zbody(i, _):
        buf_a[i >> 2, pl.ds((i & 3) * 16, 16)] = jnp.zeros((16,), jnp.float32)
        return 0
    lax.fori_loop(0, _GSZ * 4, zbody, 0)
    for k in range(5):
        pltpu.sync_copy(buf_a, acc_sh.at[pl.ds(sid * 640 + k * _GSZ, _GSZ)])
    plsc.subcore_barrier()
    pltpu.sync_copy(src2.at[wid], srci)
    pltpu.sync_copy(dst3.at[wid], dsti)

    def body(j, _):
        pltpu.sync_copy(hw.at[srci.at[j]], buf_a)
        pltpu.sync_copy(buf_a, acc_sh.at[dsti.at[j]], add=True)
        return 0
    lax.fori_loop(0, _NGRP, body, 0)
    plsc.subcore_barrier()
    pltpu.sync_copy(acc_sh.at[pl.ds(sid * 640, 640)],
                    out.at[cid, pl.ds(sid * 640, 640)])


@functools.partial(
    pl.kernel,
    out_type=jax.ShapeDtypeStruct((_NC, _PG, _H), jnp.float32),
    mesh=_mesh,
    compiler_params=_sc_params,
    scratch_types=[
        pltpu.VMEM((5, 64), jnp.int32),
        pltpu.VMEM((_RPW, _H), jnp.float32),
        pltpu.VMEM((128, _H), jnp.float32),
        pltpu.VMEM_SHARED((_PG, _H), jnp.float32),
    ],
)
def _sc_pool(h2, b3, out, bi, hrows, zr, pool_sh):
    cid = lax.axis_index("c")
    sid = lax.axis_index("s")
    wid = cid * _NS + sid
    _zero_2d(zr)

    @pl.when(sid == 0)
    def _():
        pltpu.sync_copy(zr, pool_sh.at[pl.ds(0, 128)])
        pltpu.sync_copy(zr, pool_sh.at[pl.ds(128, 128)])
        pltpu.sync_copy(zr.at[pl.ds(0, 64)], pool_sh.at[pl.ds(256, 64)])

    plsc.subcore_barrier()
    pltpu.sync_copy(b3.at[wid], bi)
    pltpu.sync_copy(h2.at[pl.ds(wid * _RPW, _RPW)], hrows)
    for k in range(5):
        pltpu.sync_copy(hrows.at[pl.ds(k * 64, 64)], pool_sh.at[bi.at[k]], add=True)
    plsc.subcore_barrier()

    @pl.when(sid == 0)
    def _():
        pltpu.sync_copy(pool_sh, out.at[cid])


def _tc1_body(x_ref, w_ref, degp_ref, hw_ref, dinv_ref):
    deg = degp_ref[0, :] + degp_ref[1, :]
    dinv = lax.rsqrt(jnp.maximum(deg, 1.0))
    dinv_ref[0, :] = dinv
    hw = jnp.dot(x_ref[...], w_ref[...], preferred_element_type=jnp.float32)
    hw_ref[...] = hw * dinv[:, None]


_tc1 = pl.pallas_call(
    _tc1_body,
    out_shape=[
        jax.ShapeDtypeStruct((_NPAD, _H), jnp.float32),
        jax.ShapeDtypeStruct((1, _NPAD), jnp.float32),
    ],
)


def _tc_mid_body(with_hs, *refs):
    if with_hs:
        pp_ref, dinv_ref, hs_ref, w_ref, sw_ref, c_ref, sc_ref, hw_ref, hsout_ref = refs
    else:
        pp_ref, dinv_ref, w_ref, sw_ref, c_ref, sc_ref, hw_ref, hsout_ref = refs
        hs_ref = None
    dinv = dinv_ref[0, :]
    h = jnp.maximum(dinv[:, None] * (pp_ref[0] + pp_ref[1]) + c_ref[0, :], 0.0)
    if hs_ref is not None:
        h = h + hs_ref[...]
    hw = jnp.dot(h, w_ref[...], preferred_element_type=jnp.float32)
    hw_ref[...] = hw * dinv[:, None]
    hsout_ref[...] = jnp.dot(h, sw_ref[...], preferred_element_type=jnp.float32) + sc_ref[0, :]


_tc_mid_shapes = [
    jax.ShapeDtypeStruct((_NPAD, _H), jnp.float32),
    jax.ShapeDtypeStruct((_NPAD, _H), jnp.float32),
]
_tc2 = pl.pallas_call(functools.partial(_tc_mid_body, False), out_shape=_tc_mid_shapes)
_tc3 = pl.pallas_call(functools.partial(_tc_mid_body, True), out_shape=_tc_mid_shapes)


def _tc4_body(pp_ref, dinv_ref, hs_ref, c_ref, h2_ref):
    dinv = dinv_ref[0, :]
    h2_ref[...] = (
        jnp.maximum(dinv[:, None] * (pp_ref[0] + pp_ref[1]) + c_ref[0, :], 0.0)
        + hs_ref[...]
    )


_tc4 = pl.pallas_call(
    _tc4_body, out_shape=jax.ShapeDtypeStruct((_NPAD, _H), jnp.float32))


def _tc5_body(poolp_ref, cntp_ref, l1w_ref, l1c_ref, l2w_ref, l2c_ref, out_ref):
    sums = poolp_ref[0] + poolp_ref[1]
    cnt = cntp_ref[0, :] + cntp_ref[1, :]
    pooled = sums[: _NG] / jnp.maximum(cnt[: _NG], 1.0)[:, None]
    z = jnp.maximum(
        jnp.dot(pooled, l1w_ref[...], preferred_element_type=jnp.float32)
        + l1c_ref[0, :], 0.0)
    out_ref[...] = (
        jnp.dot(z, l2w_ref[...], preferred_element_type=jnp.float32) + l2c_ref[0, :])


_tc5 = pl.pallas_call(
    _tc5_body, out_shape=jax.ShapeDtypeStruct((_NG, 1), jnp.float32))


def kernel(x, e_idx, b, W0, c0, W1, c1, W2, c2, S0w, S0c, S1w, S1c, L1w, L1c, L2w, L2c):
    loop = jnp.arange(_N, dtype=jnp.int32)
    pad = jnp.full((_EPAD - _ETOT,), _DUMMY, jnp.int32)
    src2 = jnp.concatenate([e_idx[0], loop, pad]).reshape(_NW, _NGRP, _GSZ)
    dst_flat = jnp.concatenate([e_idx[1], loop, pad])
    dst3 = dst_flat.reshape(_NW, _NGRP, _GSZ)
    dstc = dst_flat.reshape(_NW, _EPW // 128, 128)
    xp = jnp.concatenate([x, jnp.zeros((_NPAD - _N, _DF), jnp.float32)])
    b3 = jnp.concatenate([b, jnp.full((_NPAD - _N,), _DG, jnp.int32)]).reshape(
        _NW, 5, 64)

    degp, cntp = _sc_deg_cnt(dstc, b3)
    hw0, dinv = _tc1(xp, W0, degp)
    p0 = _sc_agg(hw0, src2, dst3)
    hw1, hs0 = _tc2(p0, dinv, W1, S0w, c0.reshape(1, _H), S0c.reshape(1, _H))
    p1 = _sc_agg(hw1, src2, dst3)
    hw2, hs1 = _tc3(p1, dinv, hs0, W2, S1w, c1.reshape(1, _H), S1c.reshape(1, _H))
    p2 = _sc_agg(hw2, src2, dst3)
    h2 = _tc4(p2, dinv, hs1, c2.reshape(1, _H))
    poolp = _sc_pool(h2, b3)
    return _tc5(poolp, cntp, L1w, L1c.reshape(1, _H // 2), L2w, L2c.reshape(1, 1))
